# baseline (speedup 1.0000x reference)
import functools

import jax
import jax.numpy as jnp
from jax import lax
from jax.experimental import pallas as pl
from jax.experimental.pallas import tpu as pltpu
from jax.experimental.pallas import tpu_sc as plsc

BSZ = 8192
IN_D = 4096
CODE_D = 8
NCODE = 96


BLK = 512


def _phase1_body(state_ref, wp_ref, bp_ref, embed_ref, embedT_ref,
                 ind_ref, loss_ref):
    h = (jnp.dot(state_ref[...], wp_ref[...],
                 preferred_element_type=jnp.float32) + bp_ref[...])
    dist = (jnp.sum(h * h, axis=1, keepdims=True)
            - 2.0 * jnp.dot(h, embed_ref[...],
                            preferred_element_type=jnp.float32)
            + jnp.sum(embed_ref[...] * embed_ref[...], axis=0, keepdims=True))
    ind = jnp.argmin(dist, axis=1).astype(jnp.int32)
    ind_ref[...] = ind
    onehot = (ind[:, None] == lax.broadcasted_iota(jnp.int32, (BLK, NCODE), 1)
              ).astype(jnp.float32)
    q = jnp.dot(onehot, embedT_ref[...], preferred_element_type=jnp.float32)
    part = jnp.sum((q - h) ** 2).reshape(1, 1)

    @pl.when(pl.program_id(0) == 0)
    def _():
        loss_ref[...] = jnp.zeros((1, 1), jnp.float32)

    loss_ref[...] += part


def _phase1(state, Wp, bp2d, embed, embedT):
    return pl.pallas_call(
        _phase1_body,
        grid=(BSZ // BLK,),
        in_specs=[
            pl.BlockSpec((BLK, IN_D), lambda i: (i, 0)),
            pl.BlockSpec((IN_D, CODE_D), lambda i: (0, 0)),
            pl.BlockSpec((1, CODE_D), lambda i: (0, 0)),
            pl.BlockSpec((CODE_D, NCODE), lambda i: (0, 0)),
            pl.BlockSpec((NCODE, CODE_D), lambda i: (0, 0)),
        ],
        out_specs=[
            pl.BlockSpec((BLK,), lambda i: (i,)),
            pl.BlockSpec((1, 1), lambda i: (0, 0)),
        ],
        out_shape=[
            jax.ShapeDtypeStruct((BSZ,), jnp.int32),
            jax.ShapeDtypeStruct((1, 1), jnp.float32),
        ],
    )(state, Wp, bp2d, embed, embedT)


def _table_body(embedT_ref, wb_ref, bb_ref, table_ref):
    table_ref[...] = (jnp.dot(embedT_ref[...], wb_ref[...],
                              preferred_element_type=jnp.float32)
                      + bb_ref[...])


def _make_table(embedT, Wb, bb2d):
    return pl.pallas_call(
        _table_body,
        out_shape=jax.ShapeDtypeStruct((NCODE, IN_D), jnp.float32),
    )(embedT, Wb, bb2d)


_NC = 2
_NS = 16
_NW = _NC * _NS
_BPW = BSZ // _NW
_CH = 8
_NCHUNK = _BPW // _CH


def _sc_gather(table, idx2d):
    mesh = plsc.VectorSubcoreMesh(core_axis_name="c", subcore_axis_name="s")

    @functools.partial(
        pl.kernel,
        mesh=mesh,
        out_type=jax.ShapeDtypeStruct((BSZ, IN_D), jnp.float32),
        scratch_types=[
            pltpu.VMEM((_NCHUNK, _CH), jnp.int32),
            pltpu.VMEM((2, _CH, IN_D), jnp.float32),
            pltpu.SemaphoreType.DMA,
            pltpu.SemaphoreType.DMA,
        ],
    )
    def gather_kernel(table_hbm, idx_hbm, out_hbm, idx_v, rows_v, sem0, sem1):
        wid = lax.axis_index("s") * _NC + lax.axis_index("c")
        chunk0 = wid * _NCHUNK
        pltpu.sync_copy(idx_hbm.at[pl.ds(chunk0, _NCHUNK)], idx_v)
        sems = [sem0, sem1]
        cps = [None, None]
        cps[0] = pltpu.async_copy(table_hbm.at[idx_v.at[0]], rows_v.at[0],
                                  sems[0])
        for j in range(_NCHUNK):
            nxt = j + 1
            if nxt < _NCHUNK:
                cps[nxt % 2] = pltpu.async_copy(
                    table_hbm.at[idx_v.at[nxt]], rows_v.at[nxt % 2],
                    sems[nxt % 2])
            cps[j % 2].wait()
            pltpu.sync_copy(rows_v.at[j % 2],
                            out_hbm.at[pl.ds((chunk0 + j) * _CH, _CH)])

    return gather_kernel(table, idx2d)


def kernel(state, Wp, bp, Wb, bb, embed):
    embedT = embed.T
    ind, loss_sum = _phase1(state, Wp, bp.reshape(1, CODE_D), embed, embedT)
    table = _make_table(embedT, Wb, bb.reshape(1, IN_D))
    out = _sc_gather(table, ind.reshape(_NW * _NCHUNK, _CH))
    cbloss = (loss_sum / (BSZ * CODE_D)).reshape(())
    att_scores = jnp.zeros((1, 1, 3), jnp.float32)
    return out, cbloss, att_scores


# speedup vs baseline: 1.3200x; 1.3200x over previous
import functools

import jax
import jax.numpy as jnp
from jax import lax
from jax.experimental import pallas as pl
from jax.experimental.pallas import tpu as pltpu
from jax.experimental.pallas import tpu_sc as plsc

BSZ = 8192
IN_D = 4096
CODE_D = 8
NCODE = 96


BLK = 512


def _phase1_body(state_ref, wp_ref, bp_ref, embed_ref, embedT_ref,
                 ind_ref, loss_ref):
    h = (jnp.dot(state_ref[...], wp_ref[...],
                 preferred_element_type=jnp.float32) + bp_ref[...])
    dist = (jnp.sum(h * h, axis=1, keepdims=True)
            - 2.0 * jnp.dot(h, embed_ref[...],
                            preferred_element_type=jnp.float32)
            + jnp.sum(embed_ref[...] * embed_ref[...], axis=0, keepdims=True))
    ind = jnp.argmin(dist, axis=1).astype(jnp.int32)
    ind_ref[...] = ind
    onehot = (ind[:, None] == lax.broadcasted_iota(jnp.int32, (BLK, NCODE), 1)
              ).astype(jnp.float32)
    q = jnp.dot(onehot, embedT_ref[...], preferred_element_type=jnp.float32)
    part = jnp.sum((q - h) ** 2).reshape(1, 1)

    @pl.when(pl.program_id(0) == 0)
    def _():
        loss_ref[...] = jnp.zeros((1, 1), jnp.float32)

    loss_ref[...] += part


def _phase1(state, Wp, bp2d, embed, embedT):
    return pl.pallas_call(
        _phase1_body,
        grid=(BSZ // BLK,),
        in_specs=[
            pl.BlockSpec((BLK, IN_D), lambda i: (i, 0)),
            pl.BlockSpec((IN_D, CODE_D), lambda i: (0, 0)),
            pl.BlockSpec((1, CODE_D), lambda i: (0, 0)),
            pl.BlockSpec((CODE_D, NCODE), lambda i: (0, 0)),
            pl.BlockSpec((NCODE, CODE_D), lambda i: (0, 0)),
        ],
        out_specs=[
            pl.BlockSpec((BLK,), lambda i: (i,)),
            pl.BlockSpec((1, 1), lambda i: (0, 0)),
        ],
        out_shape=[
            jax.ShapeDtypeStruct((BSZ,), jnp.int32),
            jax.ShapeDtypeStruct((1, 1), jnp.float32),
        ],
    )(state, Wp, bp2d, embed, embedT)


def _table_body(embedT_ref, wb_ref, bb_ref, table_ref):
    table_ref[...] = (jnp.dot(embedT_ref[...], wb_ref[...],
                              preferred_element_type=jnp.float32)
                      + bb_ref[...])


def _make_table(embedT, Wb, bb2d):
    return pl.pallas_call(
        _table_body,
        out_shape=jax.ShapeDtypeStruct((NCODE, IN_D), jnp.float32),
    )(embedT, Wb, bb2d)


_NC = 2
_NS = 16
_NW = _NC * _NS
_BPW = BSZ // _NW
_CH = 8
_NCHUNK = _BPW // _CH


def _sc_gather(table, idx2d):
    mesh = plsc.VectorSubcoreMesh(core_axis_name="c", subcore_axis_name="s")

    @functools.partial(
        pl.kernel,
        mesh=mesh,
        out_type=jax.ShapeDtypeStruct((BSZ, IN_D), jnp.float32),
        scratch_types=[
            pltpu.VMEM((_BPW,), jnp.int32),
            pltpu.VMEM_SHARED((NCODE, IN_D), jnp.float32),
            pltpu.SemaphoreType.DMA,
        ],
    )
    def gather_kernel(table_hbm, idx_hbm, out_hbm, idx_v, table_sp, sem):
        sid = lax.axis_index("s")
        wid = sid * _NC + lax.axis_index("c")
        row0 = wid * _BPW
        @pl.when(sid == 0)
        def _():
            pltpu.sync_copy(table_hbm, table_sp)

        pltpu.sync_copy(idx_hbm.at[wid], idx_v)
        plsc.subcore_barrier()

        cps = []
        for g in range(_BPW // 16):
            vals = idx_v[pl.ds(g * 16, 16)]
            for i in range(16):
                s = vals[i]
                r = g * 16 + i
                cps.append(pltpu.async_copy(
                    table_sp.at[pl.ds(s, 1)],
                    out_hbm.at[pl.ds(row0 + r, 1)], sem))
        for cp in cps:
            cp.wait()

    return gather_kernel(table, idx2d)


def kernel(state, Wp, bp, Wb, bb, embed):
    embedT = embed.T
    ind, loss_sum = _phase1(state, Wp, bp.reshape(1, CODE_D), embed, embedT)
    table = _make_table(embedT, Wb, bb.reshape(1, IN_D))
    out = _sc_gather(table, ind.reshape(_NW, _BPW))
    cbloss = (loss_sum / (BSZ * CODE_D)).reshape(())
    att_scores = jnp.zeros((1, 1, 3), jnp.float32)
    return out, cbloss, att_scores
